# baseline (speedup 1.0000x reference)
import jax
import jax.numpy as jnp
from jax import lax
from jax.experimental import pallas as pl
from jax.experimental.pallas import tpu as pltpu

N = 8192
D = 512
BLK = 512
NB = N // BLK
LAMBDA = 0.5

_CONTRACT_LAST = (((1,), (1,)), ((), ()))
_CONTRACT_INNER = (((1,), (0,)), ((), ()))


def _body(x_ref, out_ref, xscr_ref, s_ref, rinv_ref, d_ref):
    phase = pl.program_id(0)
    i = pl.program_id(1)

    @pl.when(jnp.logical_and(phase == 0, i == 0))
    def _init():
        s_ref[...] = jnp.zeros_like(s_ref)

    @pl.when(phase == 0)
    def _accumulate():
        x = x_ref[...]
        xx = x * x
        ones = jnp.ones((1, D), jnp.float32)
        q = lax.dot_general(ones, xx, _CONTRACT_LAST,
                            preferred_element_type=jnp.float32)
        rinv = lax.rsqrt(q)
        s_ref[...] += lax.dot_general(rinv, x, _CONTRACT_INNER,
                                      preferred_element_type=jnp.float32)
        rinv_ref[0, pl.ds(i * BLK, BLK)] = rinv[0, :]
        d_ref[0, pl.ds(i * BLK, BLK)] = (q * rinv * rinv)[0, :]
        xscr_ref[pl.ds(i * BLK, BLK), :] = x

    @pl.when(phase == 1)
    def _gains():
        x = xscr_ref[pl.ds(i * BLK, BLK), :]
        s = s_ref[...]
        p = lax.dot_general(s, x, _CONTRACT_LAST,
                            preferred_element_type=jnp.float32)
        rinv = rinv_ref[0, pl.ds(i * BLK, BLK)].reshape(1, BLK)
        d = d_ref[0, pl.ds(i * BLK, BLK)].reshape(1, BLK)
        out_ref[...] = p * rinv - LAMBDA * d


def kernel(X):
    out = pl.pallas_call(
        _body,
        grid=(2, NB),
        in_specs=[pl.BlockSpec((BLK, D), lambda p, i: (i * (1 - p), 0))],
        out_specs=pl.BlockSpec((1, BLK), lambda p, i: (0, i)),
        out_shape=jax.ShapeDtypeStruct((1, N), jnp.float32),
        scratch_shapes=[
            pltpu.VMEM((N, D), jnp.float32),
            pltpu.VMEM((1, D), jnp.float32),
            pltpu.VMEM((1, N), jnp.float32),
            pltpu.VMEM((1, N), jnp.float32),
        ],
    )(X)
    return out.reshape(N)


# speedup vs baseline: 1.8274x; 1.8274x over previous
import jax
import jax.numpy as jnp
from jax import lax
from jax.experimental import pallas as pl
from jax.experimental.pallas import tpu as pltpu

N = 8192
D = 512
S = 4
BLK = 512
NQ = N // S
NB = NQ // BLK
LAMBDA = 0.5

_CONTRACT_LAST = (((1,), (1,)), ((), ()))


def _body(x0_ref, x1_ref, x2_ref, x3_ref, out_ref, xn_ref, s_ref, d_ref):
    phase = pl.program_id(0)
    i = pl.program_id(1)
    x_refs = (x0_ref, x1_ref, x2_ref, x3_ref)

    @pl.when(jnp.logical_and(phase == 0, i == 0))
    def _init():
        s_ref[...] = jnp.zeros_like(s_ref)

    @pl.when(phase == 0)
    def _accumulate():
        parts = []
        for q in range(S):
            x = x_refs[q][0]
            qq = jnp.sum(x * x, axis=1, keepdims=True)
            rinv = lax.rsqrt(qq)
            xn = x * rinv
            off = q * NQ + i * BLK
            xn_ref[pl.ds(off, BLK), :] = xn
            d_ref[0, pl.ds(off, BLK)] = (qq * rinv * rinv).T[0]
            parts.append(jnp.sum(xn, axis=0, keepdims=True))
        s_ref[...] += parts[0] + parts[1] + parts[2] + parts[3]

    @pl.when(phase == 1)
    def _gains():
        s = s_ref[...]
        for q in range(S):
            off = q * NQ + i * BLK
            xn = xn_ref[pl.ds(off, BLK), :]
            p = lax.dot_general(s, xn, _CONTRACT_LAST,
                                preferred_element_type=jnp.float32)
            out_ref[0, pl.ds(off, BLK)] = p[0] - LAMBDA * d_ref[0, pl.ds(off, BLK)]


def kernel(X):
    Xr = X.reshape(S, NQ, D)
    out = pl.pallas_call(
        _body,
        grid=(2, NB),
        in_specs=[
            pl.BlockSpec((1, BLK, D), lambda p, i, q=q: (q, i * (1 - p), 0))
            for q in range(S)
        ],
        out_specs=pl.BlockSpec((1, N), lambda p, i: (0, 0)),
        out_shape=jax.ShapeDtypeStruct((1, N), jnp.float32),
        scratch_shapes=[
            pltpu.VMEM((N, D), jnp.float32),
            pltpu.VMEM((1, D), jnp.float32),
            pltpu.VMEM((1, N), jnp.float32),
        ],
    )(Xr, Xr, Xr, Xr)
    return out.reshape(N)


# speedup vs baseline: 1.8394x; 1.0066x over previous
import jax
import jax.numpy as jnp
from jax import lax
from jax.experimental import pallas as pl
from jax.experimental.pallas import tpu as pltpu

N = 8192
D = 512
S = 4
BLK = 512
NQ = N // S
NB = NQ // BLK
LAMBDA = 0.5

_CONTRACT_LAST = (((1,), (1,)), ((), ()))


def _body(x0_ref, x1_ref, x2_ref, x3_ref, out_ref, xn_ref, s_ref, d_ref):
    phase = pl.program_id(0)
    i = pl.program_id(1)
    x_refs = (x0_ref, x1_ref, x2_ref, x3_ref)

    @pl.when(jnp.logical_and(phase == 0, i == 0))
    def _init():
        s_ref[...] = jnp.zeros_like(s_ref)

    @pl.when(phase == 0)
    def _accumulate():
        parts = []
        for q in range(S):
            x = x_refs[q][0]
            qq = jnp.sum(x * x, axis=1, keepdims=True)
            rinv = lax.rsqrt(qq)
            xn = x * rinv
            off = q * NQ + i * BLK
            xn_ref[pl.ds(off, BLK), :] = xn.astype(jnp.bfloat16)
            d_ref[0, pl.ds(off, BLK)] = (qq * rinv * rinv).T[0]
            parts.append(jnp.sum(xn, axis=0, keepdims=True))
        s_ref[...] += parts[0] + parts[1] + parts[2] + parts[3]

    @pl.when(phase == 1)
    def _gains():
        s = s_ref[...].astype(jnp.bfloat16)
        for q in range(S):
            off = q * NQ + i * BLK
            xn = xn_ref[pl.ds(off, BLK), :]
            p = lax.dot_general(s, xn, _CONTRACT_LAST,
                                preferred_element_type=jnp.float32)
            out_ref[0, pl.ds(off, BLK)] = p[0] - LAMBDA * d_ref[0, pl.ds(off, BLK)]


def kernel(X):
    Xr = X.reshape(S, NQ, D)
    out = pl.pallas_call(
        _body,
        grid=(2, NB),
        in_specs=[
            pl.BlockSpec((1, BLK, D), lambda p, i, q=q: (q, i * (1 - p), 0))
            for q in range(S)
        ],
        out_specs=pl.BlockSpec((1, N), lambda p, i: (0, 0)),
        out_shape=jax.ShapeDtypeStruct((1, N), jnp.float32),
        scratch_shapes=[
            pltpu.VMEM((N, D), jnp.bfloat16),
            pltpu.VMEM((1, D), jnp.float32),
            pltpu.VMEM((1, N), jnp.float32),
        ],
    )(Xr, Xr, Xr, Xr)
    return out.reshape(N)


# speedup vs baseline: 2.0792x; 1.1304x over previous
import jax
import jax.numpy as jnp
from jax import lax
from jax.experimental import pallas as pl
from jax.experimental.pallas import tpu as pltpu

N = 8192
D = 512
S = 4
BLK = 512
NQ = N // S
NB = NQ // BLK
CHUNK = 2048
LAMBDA = 0.5

_CONTRACT_LAST = (((1,), (1,)), ((), ()))


def _body(x0_ref, x1_ref, x2_ref, x3_ref, out_ref, xn_ref, s_ref, d_ref):
    i = pl.program_id(0)
    x_refs = (x0_ref, x1_ref, x2_ref, x3_ref)

    @pl.when(i == 0)
    def _init():
        s_ref[...] = jnp.zeros_like(s_ref)

    @pl.when(i < NB)
    def _accumulate():
        parts = []
        for q in range(S):
            x = x_refs[q][0]
            qq = jnp.sum(x * x, axis=1, keepdims=True)
            rinv = lax.rsqrt(qq)
            xn = x * rinv
            off = q * NQ + i * BLK
            xn_ref[pl.ds(off, BLK), :] = xn.astype(jnp.bfloat16)
            d_ref[0, pl.ds(off, BLK)] = (qq * rinv * rinv).T[0]
            parts.append(jnp.sum(xn, axis=0, keepdims=True))
        s_ref[...] += parts[0] + parts[1] + parts[2] + parts[3]

    @pl.when(i == NB)
    def _gains():
        s = s_ref[...].astype(jnp.bfloat16)
        for c in range(N // CHUNK):
            xn = xn_ref[pl.ds(c * CHUNK, CHUNK), :]
            p = lax.dot_general(s, xn, _CONTRACT_LAST,
                                preferred_element_type=jnp.float32)
            out_ref[0, pl.ds(c * CHUNK, CHUNK)] = (
                p[0] - LAMBDA * d_ref[0, pl.ds(c * CHUNK, CHUNK)])


def kernel(X):
    Xr = X.reshape(S, NQ, D)
    out = pl.pallas_call(
        _body,
        grid=(NB + 1,),
        in_specs=[
            pl.BlockSpec((1, BLK, D),
                         lambda i, q=q: (q, jnp.minimum(i, NB - 1), 0))
            for q in range(S)
        ],
        out_specs=pl.BlockSpec((1, N), lambda i: (0, 0)),
        out_shape=jax.ShapeDtypeStruct((1, N), jnp.float32),
        scratch_shapes=[
            pltpu.VMEM((N, D), jnp.bfloat16),
            pltpu.VMEM((1, D), jnp.float32),
            pltpu.VMEM((1, N), jnp.float32),
        ],
    )(Xr, Xr, Xr, Xr)
    return out.reshape(N)


# speedup vs baseline: 2.2119x; 1.0638x over previous
import jax
import jax.numpy as jnp
from jax import lax
from jax.experimental import pallas as pl
from jax.experimental.pallas import tpu as pltpu

N = 8192
D = 512
S = 4
BLK = 512
NQ = N // S
NB = NQ // BLK
CHUNK = 2048
LAMBDA = 0.5

_CONTRACT_LAST = (((1,), (1,)), ((), ()))


def _body(x0_ref, x1_ref, x2_ref, x3_ref, out_ref, xn_ref, s_ref):
    i = pl.program_id(0)
    x_refs = (x0_ref, x1_ref, x2_ref, x3_ref)

    @pl.when(i == 0)
    def _init():
        s_ref[...] = jnp.zeros_like(s_ref)

    @pl.when(i < NB)
    def _accumulate():
        parts = []
        for q in range(S):
            x = x_refs[q][0]
            qq = jnp.sum(x * x, axis=1, keepdims=True)
            rinv = lax.rsqrt(qq)
            xn = x * rinv
            off = q * NQ + i * BLK
            xn_ref[pl.ds(off, BLK), :] = xn.astype(jnp.bfloat16)
            parts.append(jnp.sum(xn, axis=0, keepdims=True))
        s_ref[...] += parts[0] + parts[1] + parts[2] + parts[3]

    @pl.when(i == NB)
    def _gains():
        s = s_ref[...].astype(jnp.bfloat16)
        for c in range(N // CHUNK):
            xn = xn_ref[pl.ds(c * CHUNK, CHUNK), :]
            p = lax.dot_general(s, xn, _CONTRACT_LAST,
                                preferred_element_type=jnp.float32)
            out_ref[0, pl.ds(c * CHUNK, CHUNK)] = p[0] - LAMBDA


def kernel(X):
    Xr = X.reshape(S, NQ, D)
    out = pl.pallas_call(
        _body,
        grid=(NB + 1,),
        in_specs=[
            pl.BlockSpec((1, BLK, D),
                         lambda i, q=q: (q, jnp.minimum(i, NB - 1), 0))
            for q in range(S)
        ],
        out_specs=pl.BlockSpec((1, N), lambda i: (0, 0)),
        out_shape=jax.ShapeDtypeStruct((1, N), jnp.float32),
        scratch_shapes=[
            pltpu.VMEM((N, D), jnp.bfloat16),
            pltpu.VMEM((1, D), jnp.float32),
        ],
    )(Xr, Xr, Xr, Xr)
    return out.reshape(N)
